# baseline (speedup 1.0000x reference)
import jax
import jax.numpy as jnp
from jax import lax
from jax.experimental import pallas as pl
from jax.experimental.pallas import tpu as pltpu
from jax.experimental.pallas import tpu_sc as plsc


def _sc_strips(c_sh, nh, n):
    def body(c_hbm, strip_hbm, s_v, sem):
        cid = lax.axis_index("c")
        sid = lax.axis_index("s")
        wid = sid * 2 + cid
        h = wid // 2
        half = wid % 2
        builds = []
        for i in range(n // 2):
            pw = half * (n // 2) + i
            t = 31 - pw
            smod = (31 - i) % 8
            base = pl.multiple_of(t - smod, 8)
            builds.append(
                pltpu.async_copy(
                    c_hbm.at[smod, h, :, pl.ds(base, n)],
                    s_v.at[i],
                    sem,
                )
            )
        for b in builds:
            b.wait()
        dumps = []
        for i in range(n // 2):
            pw = half * (n // 2) + i
            dumps.append(
                pltpu.async_copy(s_v.at[i], strip_hbm.at[h, pw], sem)
            )
        for b in dumps:
            b.wait()

    run = pl.kernel(
        body,
        out_type=jax.ShapeDtypeStruct((nh, n, 63, n), jnp.float32),
        mesh=plsc.VectorSubcoreMesh(core_axis_name="c", subcore_axis_name="s"),
        scratch_types=[
            pltpu.VMEM((n // 2, 63, n), jnp.float32),
            pltpu.SemaphoreType.DMA,
        ],
        compiler_params=pltpu.CompilerParams(use_tc_tiling_on_sc=False),
    )
    return run(c_sh)


def _tc_expand(strip2, nh, n):
    hw = n * n
    w = hw + 128

    def body(s_ref, o_ref):
        ph = pl.program_id(1)
        t = 31 - ph
        base = pl.multiple_of((t // 4) * 128, 128)
        rem = (t % 4) * n
        x = s_ref[0, :, pl.ds(base, w)]
        y = pltpu.roll(x, w - rem, axis=1)
        o_ref[0, 0] = y[:, :hw]

    return pl.pallas_call(
        body,
        grid=(nh, n),
        in_specs=[
            pl.BlockSpec((1, n, 2048), lambda h, ph: (h, 0, 0)),
        ],
        out_specs=pl.BlockSpec((1, 1, n, hw), lambda h, ph: (0, h, ph, 0)),
        out_shape=jax.ShapeDtypeStruct((1, nh, hw, hw), jnp.float32),
    )(strip2)


def kernel(H, W, relative_position_bias_table):
    table = relative_position_bias_table
    nh = table.shape[1]
    side = int(round(table.shape[0] ** 0.5))
    n = (side + 1) // 2
    off = (jnp.asarray(H, jnp.int32) - n) + (jnp.asarray(W, jnp.int32) - n)
    table = jnp.roll(table, -off, axis=0)
    c = jnp.transpose(table.reshape(side, side, nh)[::-1, ::-1, :], (2, 0, 1))
    c_wide = jnp.pad(c, ((0, 0), (0, 0), (0, 72 - side)))
    c_sh = jnp.stack([c_wide[:, :, s:s + 64] for s in range(8)])
    strips = _sc_strips(c_sh, nh, n)
    strip2 = jnp.pad(strips.reshape(nh, n, 63 * n), ((0, 0), (0, 0), (0, 32)))
    return _tc_expand(strip2, nh, n)


# speedup vs baseline: 2.9547x; 2.9547x over previous
import jax
import jax.numpy as jnp
from jax import lax
from jax.experimental import pallas as pl
from jax.experimental.pallas import tpu as pltpu
from jax.experimental.pallas import tpu_sc as plsc


def _sc_strips(c_sh, nh, n):
    def body(c_hbm, strip_hbm, s_v, sem):
        cid = lax.axis_index("c")
        sid = lax.axis_index("s")
        wid = sid * 2 + cid
        h = wid // 2
        half = wid % 2
        builds = []
        for i in range(n // 2):
            pw = half * (n // 2) + i
            t = 31 - pw
            smod = (31 - i) % 8
            base = pl.multiple_of(t - smod, 8)
            builds.append(
                pltpu.async_copy(
                    c_hbm.at[smod, h, :, pl.ds(base, n)],
                    s_v.at[i],
                    sem,
                )
            )
        for b in builds:
            b.wait()
        dumps = []
        for i in range(n // 2):
            pw = half * (n // 2) + i
            dumps.append(
                pltpu.async_copy(s_v.at[i], strip_hbm.at[h, pw], sem)
            )
        for b in dumps:
            b.wait()

    run = pl.kernel(
        body,
        out_type=jax.ShapeDtypeStruct((nh, n, 63, n), jnp.float32),
        mesh=plsc.VectorSubcoreMesh(core_axis_name="c", subcore_axis_name="s"),
        scratch_types=[
            pltpu.VMEM((n // 2, 63, n), jnp.float32),
            pltpu.SemaphoreType.DMA,
        ],
        compiler_params=pltpu.CompilerParams(use_tc_tiling_on_sc=False),
    )
    return run(c_sh)


def _tc_expand(strip2, nh, n):
    hw = n * n
    w = hw + 128

    def body(s_ref, o_ref):
        pid = pl.program_id(1)
        for j in range(8):
            ph = pid * 8 + j
            t = 31 - ph
            base = pl.multiple_of((t // 4) * 128, 128)
            rem = ((3 - j) % 4) * n
            x = s_ref[0, :, pl.ds(base, w)]
            if rem:
                x = pltpu.roll(x, w - rem, axis=1)
            o_ref[0, 0, j * n:(j + 1) * n, :] = x[:, :hw]

    return pl.pallas_call(
        body,
        grid=(nh, 4),
        in_specs=[
            pl.BlockSpec((1, n, 2048), lambda h, pid: (h, 0, 0)),
        ],
        out_specs=pl.BlockSpec((1, 1, 8 * n, hw), lambda h, pid: (0, h, pid, 0)),
        out_shape=jax.ShapeDtypeStruct((1, nh, hw, hw), jnp.float32),
    )(strip2)


def kernel(H, W, relative_position_bias_table):
    table = relative_position_bias_table
    nh = table.shape[1]
    side = int(round(table.shape[0] ** 0.5))
    n = (side + 1) // 2
    off = (jnp.asarray(H, jnp.int32) - n) + (jnp.asarray(W, jnp.int32) - n)
    table = jnp.roll(table, -off, axis=0)
    c = jnp.transpose(table.reshape(side, side, nh)[::-1, ::-1, :], (2, 0, 1))
    c_wide = jnp.pad(c, ((0, 0), (0, 0), (0, 72 - side)))
    c_sh = jnp.stack([c_wide[:, :, s:s + 64] for s in range(8)])
    strips = _sc_strips(c_sh, nh, n)
    strip2 = jnp.pad(strips.reshape(nh, n, 63 * n), ((0, 0), (0, 0), (0, 32)))
    return _tc_expand(strip2, nh, n)


# speedup vs baseline: 3.0232x; 1.0232x over previous
import jax
import jax.numpy as jnp
from jax import lax
from jax.experimental import pallas as pl
from jax.experimental.pallas import tpu as pltpu
from jax.experimental.pallas import tpu_sc as plsc


def _sc_strips(c_sh, nh, n):
    def body(c_hbm, strip_hbm, s_v, sem):
        cid = lax.axis_index("c")
        sid = lax.axis_index("s")
        wid = sid * 2 + cid
        h = wid // 2
        half = wid % 2
        builds = []
        for i in range(n // 2):
            pw = half * (n // 2) + i
            t = 31 - pw
            smod = (31 - i) % 8
            base = pl.multiple_of(t - smod, 8)
            builds.append(
                pltpu.async_copy(
                    c_hbm.at[smod, h, :, pl.ds(base, n)],
                    s_v.at[i, pl.ds(0, 63), :],
                    sem,
                )
            )
        for b in builds:
            b.wait()
        dumps = []
        for i in range(n // 2):
            pw = half * (n // 2) + i
            dumps.append(
                pltpu.async_copy(s_v.at[i], strip_hbm.at[h, pw], sem)
            )
        for b in dumps:
            b.wait()

    run = pl.kernel(
        body,
        out_type=jax.ShapeDtypeStruct((nh, n, 64, n), jnp.float32),
        mesh=plsc.VectorSubcoreMesh(core_axis_name="c", subcore_axis_name="s"),
        scratch_types=[
            pltpu.VMEM((n // 2, 64, n), jnp.float32),
            pltpu.SemaphoreType.DMA,
        ],
        compiler_params=pltpu.CompilerParams(use_tc_tiling_on_sc=False),
    )
    return run(c_sh)


def _tc_expand(strip2, nh, n):
    hw = n * n
    w = hw + 128

    def body(s_ref, o_ref):
        pid = pl.program_id(1)
        for j in range(8):
            ph = pid * 8 + j
            t = 31 - ph
            base = pl.multiple_of((t // 4) * 128, 128)
            rem = ((3 - j) % 4) * n
            x = s_ref[0, :, pl.ds(base, w)]
            if rem:
                x = pltpu.roll(x, w - rem, axis=1)
            o_ref[0, 0, j * n:(j + 1) * n, :] = x[:, :hw]

    return pl.pallas_call(
        body,
        grid=(nh, 4),
        in_specs=[
            pl.BlockSpec((1, n, 2048), lambda h, pid: (h, 0, 0)),
        ],
        out_specs=pl.BlockSpec((1, 1, 8 * n, hw), lambda h, pid: (0, h, pid, 0)),
        out_shape=jax.ShapeDtypeStruct((1, nh, hw, hw), jnp.float32),
    )(strip2)


def kernel(H, W, relative_position_bias_table):
    table = relative_position_bias_table
    nh = table.shape[1]
    side = int(round(table.shape[0] ** 0.5))
    n = (side + 1) // 2
    off = (jnp.asarray(H, jnp.int32) - n) + (jnp.asarray(W, jnp.int32) - n)
    table = jnp.roll(table, -off, axis=0)
    c = jnp.transpose(table.reshape(side, side, nh)[::-1, ::-1, :], (2, 0, 1))
    c_wide = jnp.pad(c, ((0, 0), (0, 0), (0, 72 - side)))
    c_sh = jnp.stack([c_wide[:, :, s:s + 64] for s in range(8)])
    strips = _sc_strips(c_sh, nh, n)
    return _tc_expand(strips.reshape(nh, n, 64 * n), nh, n)


# speedup vs baseline: 3.6231x; 1.1985x over previous
import jax
import jax.numpy as jnp
from jax import lax
from jax.experimental import pallas as pl
from jax.experimental.pallas import tpu as pltpu
from jax.experimental.pallas import tpu_sc as plsc


def _sc_strips(c_sh, nh, n):
    def body(c_hbm, strip_hbm, s_v, sem):
        cid = lax.axis_index("c")
        sid = lax.axis_index("s")
        wid = sid * 2 + cid
        h = wid // 2
        half = wid % 2
        builds = []
        for i in range(n // 2):
            pw = half * (n // 2) + i
            t = 31 - pw
            smod = (31 - i) % 8
            base = pl.multiple_of(t - smod, 8)
            builds.append(
                pltpu.async_copy(
                    c_hbm.at[smod, h, :, pl.ds(base, n)],
                    s_v.at[i, pl.ds(0, 63), :],
                    sem,
                )
            )
        for b in builds:
            b.wait()
        dumps = []
        for i in range(n // 2):
            pw = half * (n // 2) + i
            dumps.append(
                pltpu.async_copy(s_v.at[i], strip_hbm.at[h, pw], sem)
            )
        for b in dumps:
            b.wait()

    run = pl.kernel(
        body,
        out_type=jax.ShapeDtypeStruct((nh, n, 64, n), jnp.float32),
        mesh=plsc.VectorSubcoreMesh(core_axis_name="c", subcore_axis_name="s"),
        scratch_types=[
            pltpu.VMEM((n // 2, 64, n), jnp.float32),
            pltpu.SemaphoreType.DMA,
        ],
        compiler_params=pltpu.CompilerParams(use_tc_tiling_on_sc=False),
    )
    return run(c_sh)


def _tc_expand(strip2, nh, n):
    hw = n * n
    w = hw + 128

    def body(s_ref, o_ref):
        pid = pl.program_id(1)
        for j in range(16):
            ph = pid * 16 + j
            t = 31 - ph
            base = pl.multiple_of((t // 4) * 128, 128)
            rem = ((3 - j) % 4) * n
            x = s_ref[0, :, pl.ds(base, w)]
            if rem:
                x = pltpu.roll(x, w - rem, axis=1)
            o_ref[0, 0, j * n:(j + 1) * n, :] = x[:, :hw]

    return pl.pallas_call(
        body,
        grid=(nh, 2),
        in_specs=[
            pl.BlockSpec((1, n, 2048), lambda h, pid: (h, 0, 0)),
        ],
        out_specs=pl.BlockSpec((1, 1, 16 * n, hw), lambda h, pid: (0, h, pid, 0)),
        out_shape=jax.ShapeDtypeStruct((1, nh, hw, hw), jnp.float32),
    )(strip2)


def kernel(H, W, relative_position_bias_table):
    table = relative_position_bias_table
    nh = table.shape[1]
    side = int(round(table.shape[0] ** 0.5))
    n = (side + 1) // 2
    off = (jnp.asarray(H, jnp.int32) - n) + (jnp.asarray(W, jnp.int32) - n)
    table = jnp.roll(table, -off, axis=0)
    c = jnp.transpose(table.reshape(side, side, nh)[::-1, ::-1, :], (2, 0, 1))
    c_wide = jnp.pad(c, ((0, 0), (0, 0), (0, 72 - side)))
    c_sh = jnp.stack([c_wide[:, :, s:s + 64] for s in range(8)])
    strips = _sc_strips(c_sh, nh, n)
    return _tc_expand(strips.reshape(nh, n, 64 * n), nh, n)


# speedup vs baseline: 3.9704x; 1.0958x over previous
import jax
import jax.numpy as jnp
from jax import lax
from jax.experimental import pallas as pl
from jax.experimental.pallas import tpu as pltpu
from jax.experimental.pallas import tpu_sc as plsc


def _sc_strips(c_sh, nh, n):
    def body(c_hbm, strip_hbm, s_v, sem):
        cid = lax.axis_index("c")
        sid = lax.axis_index("s")
        wid = sid * 2 + cid
        h = wid // 2
        half = wid % 2
        builds = []
        for i in range(n // 2):
            pw = half * (n // 2) + i
            t = 31 - pw
            smod = (31 - i) % 8
            base = pl.multiple_of(t - smod, 8)
            builds.append(
                pltpu.async_copy(
                    c_hbm.at[smod, h, :, pl.ds(base, n)],
                    s_v.at[i, pl.ds(0, 63), :],
                    sem,
                )
            )
        for b in builds:
            b.wait()
        dumps = []
        for i in range(n // 2):
            pw = half * (n // 2) + i
            dumps.append(
                pltpu.async_copy(s_v.at[i], strip_hbm.at[h, pw], sem)
            )
        for b in dumps:
            b.wait()

    run = pl.kernel(
        body,
        out_type=jax.ShapeDtypeStruct((nh, n, 64, n), jnp.float32),
        mesh=plsc.VectorSubcoreMesh(core_axis_name="c", subcore_axis_name="s"),
        scratch_types=[
            pltpu.VMEM((n // 2, 64, n), jnp.float32),
            pltpu.SemaphoreType.DMA,
        ],
        compiler_params=pltpu.CompilerParams(use_tc_tiling_on_sc=False),
    )
    return run(c_sh)


def _tc_expand(strip2, nh, n):
    hw = n * n
    w = hw + 128

    def body(s_ref, o_ref):
        for j in range(32):
            ph = j
            t = 31 - ph
            base = pl.multiple_of((t // 4) * 128, 128)
            rem = ((3 - j) % 4) * n
            x = s_ref[0, :, pl.ds(base, w)]
            if rem:
                x = pltpu.roll(x, w - rem, axis=1)
            o_ref[0, 0, j * n:(j + 1) * n, :] = x[:, :hw]

    return pl.pallas_call(
        body,
        grid=(nh,),
        in_specs=[
            pl.BlockSpec((1, n, 2048), lambda h: (h, 0, 0)),
        ],
        out_specs=pl.BlockSpec((1, 1, hw, hw), lambda h: (0, h, 0, 0)),
        out_shape=jax.ShapeDtypeStruct((1, nh, hw, hw), jnp.float32),
    )(strip2)


def kernel(H, W, relative_position_bias_table):
    table = relative_position_bias_table
    nh = table.shape[1]
    side = int(round(table.shape[0] ** 0.5))
    n = (side + 1) // 2
    off = (jnp.asarray(H, jnp.int32) - n) + (jnp.asarray(W, jnp.int32) - n)
    table = jnp.roll(table, -off, axis=0)
    c = jnp.transpose(table.reshape(side, side, nh)[::-1, ::-1, :], (2, 0, 1))
    c_wide = jnp.pad(c, ((0, 0), (0, 0), (0, 72 - side)))
    c_sh = jnp.stack([c_wide[:, :, s:s + 64] for s in range(8)])
    strips = _sc_strips(c_sh, nh, n)
    return _tc_expand(strips.reshape(nh, n, 64 * n), nh, n)


# speedup vs baseline: 4.0458x; 1.0190x over previous
import jax
import jax.numpy as jnp
from jax import lax
from jax.experimental import pallas as pl
from jax.experimental.pallas import tpu as pltpu
from jax.experimental.pallas import tpu_sc as plsc


def _sc_strips(c_sh, nh, n):
    def body(c_hbm, strip_hbm, s_v, sem):
        cid = lax.axis_index("c")
        sid = lax.axis_index("s")
        wid = sid * 2 + cid
        h = wid // 2
        half = wid % 2
        builds = []
        for i in range(n // 2):
            pw = half * (n // 2) + i
            t = 31 - pw
            smod = (31 - i) % 8
            base = pl.multiple_of(t - smod, 8)
            builds.append(
                pltpu.async_copy(
                    c_hbm.at[smod, h, :, pl.ds(base, n)],
                    s_v.at[i, pl.ds(0, 63), :],
                    sem,
                )
            )
        for b in builds:
            b.wait()
        dumps = []
        for i in range(n // 2):
            pw = half * (n // 2) + i
            dumps.append(
                pltpu.async_copy(s_v.at[i], strip_hbm.at[h, pw], sem)
            )
        for b in dumps:
            b.wait()

    run = pl.kernel(
        body,
        out_type=jax.ShapeDtypeStruct((nh, n, 64, n), jnp.float32),
        mesh=plsc.VectorSubcoreMesh(core_axis_name="c", subcore_axis_name="s"),
        scratch_types=[
            pltpu.VMEM((n // 2, 64, n), jnp.float32),
            pltpu.SemaphoreType.DMA,
        ],
        compiler_params=pltpu.CompilerParams(use_tc_tiling_on_sc=False),
    )
    return run(c_sh)


def _tc_expand(strip2, nh, n):
    hw = n * n
    w = hw + 128

    def body(s_ref, o_ref):
        h = pl.program_id(0)
        for j in range(32):
            ph = j
            t = 31 - ph
            base = pl.multiple_of((t // 4) * 128, 128)
            rem = ((3 - j) % 4) * n
            x = s_ref[h, :, pl.ds(base, w)]
            if rem:
                x = pltpu.roll(x, w - rem, axis=1)
            o_ref[0, 0, j * n:(j + 1) * n, :] = x[:, :hw]

    return pl.pallas_call(
        body,
        grid=(nh,),
        in_specs=[
            pl.BlockSpec((nh, n, 2048), lambda h: (0, 0, 0)),
        ],
        out_specs=pl.BlockSpec((1, 1, hw, hw), lambda h: (0, h, 0, 0)),
        out_shape=jax.ShapeDtypeStruct((1, nh, hw, hw), jnp.float32),
    )(strip2)


def kernel(H, W, relative_position_bias_table):
    table = relative_position_bias_table
    nh = table.shape[1]
    side = int(round(table.shape[0] ** 0.5))
    n = (side + 1) // 2
    off = (jnp.asarray(H, jnp.int32) - n) + (jnp.asarray(W, jnp.int32) - n)
    table = jnp.roll(table, -off, axis=0)
    c = jnp.transpose(table.reshape(side, side, nh)[::-1, ::-1, :], (2, 0, 1))
    c_wide = jnp.pad(c, ((0, 0), (0, 0), (0, 72 - side)))
    c_sh = jnp.stack([c_wide[:, :, s:s + 64] for s in range(8)])
    strips = _sc_strips(c_sh, nh, n)
    return _tc_expand(strips.reshape(nh, n, 64 * n), nh, n)
